# baseline (speedup 1.0000x reference)
import jax
import jax.numpy as jnp
from jax.experimental import pallas as pl
from jax.experimental.pallas import tpu as pltpu


def _body(lap_ref, fk_ref, fm_ref, w1t_ref, w2t_ref, b_ref, out_ref):
    x = jnp.dot(lap_ref[...].astype(jnp.bfloat16), fk_ref[...],
                preferred_element_type=jnp.float32)
    f = fm_ref[...]
    out_ref[...] = (
        jnp.dot(f + x, w1t_ref[...], preferred_element_type=jnp.float32)
        + jnp.dot(x * f, w2t_ref[...], preferred_element_type=jnp.float32)
        + b_ref[...]
    )


def kernel(lap_matrix, eye_matrix, features, W1, b1, W2, b2):
    del eye_matrix
    n, d = features.shape
    bm = 200

    feat_bf = features.astype(jnp.bfloat16)
    w1t = W1.T
    w2t = W2.T
    bias = (b1 + b2).reshape(1, d)

    grid = (n // bm,)
    return pl.pallas_call(
        _body,
        grid=grid,
        in_specs=[
            pl.BlockSpec((bm, n), lambda m: (m, 0)),
            pl.BlockSpec((n, d), lambda m: (0, 0)),
            pl.BlockSpec((bm, d), lambda m: (m, 0)),
            pl.BlockSpec((d, d), lambda m: (0, 0)),
            pl.BlockSpec((d, d), lambda m: (0, 0)),
            pl.BlockSpec((1, d), lambda m: (0, 0)),
        ],
        out_specs=pl.BlockSpec((bm, d), lambda m: (m, 0)),
        out_shape=jax.ShapeDtypeStruct((n, d), jnp.float32),
        compiler_params=pltpu.CompilerParams(
            dimension_semantics=("arbitrary",),
        ),
    )(lap_matrix, feat_bf, features, w1t, w2t, bias)


# speedup vs baseline: 1.0127x; 1.0127x over previous
import jax
import jax.numpy as jnp
from jax.experimental import pallas as pl
from jax.experimental.pallas import tpu as pltpu


def _body(lap_ref, fk_ref, fm_ref, w1t_ref, w2t_ref, b_ref, out_ref):
    x = jnp.dot(lap_ref[...].astype(jnp.bfloat16), fk_ref[...],
                preferred_element_type=jnp.float32)
    f = fm_ref[...]
    out_ref[...] = (
        jnp.dot(f + x, w1t_ref[...], preferred_element_type=jnp.float32)
        + jnp.dot(x * f, w2t_ref[...], preferred_element_type=jnp.float32)
        + b_ref[...]
    )


def kernel(lap_matrix, eye_matrix, features, W1, b1, W2, b2):
    del eye_matrix
    n, d = features.shape
    bm = 400

    feat_bf = features.astype(jnp.bfloat16)
    w1t = W1.T
    w2t = W2.T
    bias = (b1 + b2).reshape(1, d)

    grid = (n // bm,)
    return pl.pallas_call(
        _body,
        grid=grid,
        in_specs=[
            pl.BlockSpec((bm, n), lambda m: (m, 0)),
            pl.BlockSpec((n, d), lambda m: (0, 0)),
            pl.BlockSpec((bm, d), lambda m: (m, 0)),
            pl.BlockSpec((d, d), lambda m: (0, 0)),
            pl.BlockSpec((d, d), lambda m: (0, 0)),
            pl.BlockSpec((1, d), lambda m: (0, 0)),
        ],
        out_specs=pl.BlockSpec((bm, d), lambda m: (m, 0)),
        out_shape=jax.ShapeDtypeStruct((n, d), jnp.float32),
        compiler_params=pltpu.CompilerParams(
            dimension_semantics=("arbitrary",),
        ),
    )(lap_matrix, feat_bf, features, w1t, w2t, bias)


# speedup vs baseline: 1.0349x; 1.0219x over previous
import jax
import jax.numpy as jnp
from jax.experimental import pallas as pl
from jax.experimental.pallas import tpu as pltpu


def _body(l1_ref, l2_ref, fk_ref, fm_ref, w1t_ref, w2t_ref, b_ref, out_ref):
    hm = l1_ref.shape[0]
    fk = fk_ref[...]
    w1t = w1t_ref[...]
    w2t = w2t_ref[...]
    b = b_ref[...]
    for i, l_ref in enumerate((l1_ref, l2_ref)):
        sl = pl.ds(i * hm, hm)
        x = jnp.dot(l_ref[...].astype(jnp.bfloat16), fk,
                    preferred_element_type=jnp.float32)
        f = fm_ref[sl, :]
        out_ref[sl, :] = (
            jnp.dot(f + x, w1t, preferred_element_type=jnp.float32)
            + jnp.dot(x * f, w2t, preferred_element_type=jnp.float32)
            + b
        )


def kernel(lap_matrix, eye_matrix, features, W1, b1, W2, b2):
    del eye_matrix
    n, d = features.shape
    bm = 400
    hm = bm // 2

    feat_bf = features.astype(jnp.bfloat16)
    w1t = W1.T
    w2t = W2.T
    bias = (b1 + b2).reshape(1, d)

    grid = (n // bm,)
    return pl.pallas_call(
        _body,
        grid=grid,
        in_specs=[
            pl.BlockSpec((hm, n), lambda m: (2 * m, 0)),
            pl.BlockSpec((hm, n), lambda m: (2 * m + 1, 0)),
            pl.BlockSpec((n, d), lambda m: (0, 0)),
            pl.BlockSpec((bm, d), lambda m: (m, 0)),
            pl.BlockSpec((d, d), lambda m: (0, 0)),
            pl.BlockSpec((d, d), lambda m: (0, 0)),
            pl.BlockSpec((1, d), lambda m: (0, 0)),
        ],
        out_specs=pl.BlockSpec((bm, d), lambda m: (m, 0)),
        out_shape=jax.ShapeDtypeStruct((n, d), jnp.float32),
        compiler_params=pltpu.CompilerParams(
            dimension_semantics=("arbitrary",),
        ),
    )(lap_matrix, lap_matrix, feat_bf, features, w1t, w2t, bias)
